# baseline (speedup 1.0000x reference)
import jax
import jax.numpy as jnp
from jax import lax
from jax.experimental import pallas as pl
from jax.experimental.pallas import tpu as pltpu
from jax.experimental.pallas import tpu_sc as plsc

D_MODEL = 64
BATCH = 4096
SAMPLE = 200
NUM_CORES = 2
NUM_SUBCORES = 16
NUM_WORKERS = NUM_CORES * NUM_SUBCORES
BPW = BATCH // NUM_WORKERS
HALF = SAMPLE // 2
SPAD = 208
NGROUPS = SPAD // 16
LANES = 16


def _tec_body(h_hbm, idx_hbm, tbl_hbm, out_hbm,
              idx_v, h_v, rows0, rows1, out0, out1, tr_v,
              gsem0, gsem1, osem0, osem1):
  cid = lax.axis_index("c")
  sid = lax.axis_index("s")
  wid = sid * NUM_CORES + cid
  b0 = wid * BPW

  pltpu.sync_copy(idx_hbm.at[pl.ds(wid * 2 * BPW, 2 * BPW)], idx_v)
  pltpu.sync_copy(h_hbm.at[pl.ds(b0, BPW)], h_v)

  rows = (rows0, rows1)
  outs = (out0, out1)
  gsems = (gsem0, gsem1)
  osems = (osem0, osem1)

  iota = lax.iota(jnp.int32, LANES)
  col_ids = [jnp.full((LANES,), l, jnp.int32) for l in range(LANES)]

  def fire_gather(i, p):
    pltpu.async_copy(tbl_hbm.at[idx_v.at[2 * i]],
                     rows[p].at[pl.ds(0, HALF)], gsems[p])
    pltpu.async_copy(tbl_hbm.at[idx_v.at[2 * i + 1]],
                     rows[p].at[pl.ds(HALF, HALF)], gsems[p])

  def wait_gather(i, p):
    pltpu.make_async_copy(tbl_hbm.at[idx_v.at[2 * i]],
                          rows[p].at[pl.ds(0, HALF)], gsems[p]).wait()
    pltpu.make_async_copy(tbl_hbm.at[idx_v.at[2 * i + 1]],
                          rows[p].at[pl.ds(HALF, HALF)], gsems[p]).wait()

  def out_desc(i, p):
    return pltpu.make_async_copy(outs[p].at[pl.ds(0, SAMPLE)],
                                 out_hbm.at[b0 + i], osems[p])

  fire_gather(0, 0)
  fire_gather(1, 1)

  def compute(i, p):
    hv = [h_v[i, pl.ds(16 * j, 16)] for j in range(4)]
    r = rows[p]
    for g in range(NGROUPS):
      base = 16 * g
      for rr in range(16):
        row = base + rr
        part = r[row, pl.ds(0, 16)] * hv[0]
        part = part + r[row, pl.ds(16, 16)] * hv[1]
        part = part + r[row, pl.ds(32, 16)] * hv[2]
        part = part + r[row, pl.ds(48, 16)] * hv[3]
        tr_v[rr, pl.ds(0, 16)] = part
      acc = plsc.load_gather(tr_v, [iota, col_ids[0]])
      for l in range(1, 16):
        acc = acc + plsc.load_gather(tr_v, [iota, col_ids[l]])
      outs[p][pl.ds(base, 16)] = acc

  def step(k, carry):
    for p in range(2):
      i = 2 * k + p
      wait_gather(i, p)

      @pl.when(i >= 2)
      def _():
        out_desc(i - 2, p).wait()

      compute(i, p)
      pltpu.async_copy(outs[p].at[pl.ds(0, SAMPLE)],
                       out_hbm.at[b0 + i], osems[p])

      @pl.when(i + 2 < BPW)
      def _():
        fire_gather(i + 2, p)
    return carry

  lax.fori_loop(0, BPW // 2, step, 0)
  out_desc(BPW - 2, 0).wait()
  out_desc(BPW - 1, 1).wait()


_sc_call = pl.kernel(
    _tec_body,
    out_type=jax.ShapeDtypeStruct((BATCH, SAMPLE), jnp.float32),
    mesh=plsc.VectorSubcoreMesh(core_axis_name="c", subcore_axis_name="s",
                                num_cores=NUM_CORES,
                                num_subcores=NUM_SUBCORES),
    compiler_params=pltpu.CompilerParams(needs_layout_passes=False,
                                         use_tc_tiling_on_sc=False),
    scratch_types=[
        pltpu.VMEM((2 * BPW, HALF), jnp.int32),
        pltpu.VMEM((BPW, D_MODEL), jnp.float32),
        pltpu.VMEM((SPAD, D_MODEL), jnp.float32),
        pltpu.VMEM((SPAD, D_MODEL), jnp.float32),
        pltpu.VMEM((SPAD,), jnp.float32),
        pltpu.VMEM((SPAD,), jnp.float32),
        pltpu.VMEM((LANES, LANES), jnp.float32),
        pltpu.SemaphoreType.DMA,
        pltpu.SemaphoreType.DMA,
        pltpu.SemaphoreType.DMA,
        pltpu.SemaphoreType.DMA,
    ],
)


@jax.jit
def _run(h2, idx2, tbl):
  return _sc_call(h2, idx2, tbl)


def kernel(h, indicies, emb_weight):
  h2 = jnp.reshape(h, (BATCH, D_MODEL))
  idx2 = jnp.reshape(jnp.asarray(indicies, jnp.int32),
                     (BATCH * SAMPLE // HALF, HALF))
  out = _run(h2, idx2, emb_weight)
  return jnp.reshape(out, (BATCH, 1, SAMPLE))
